# baseline (speedup 1.0000x reference)
import functools

import jax
import jax.numpy as jnp
from jax import lax
from jax.experimental import pallas as pl
from jax.experimental.pallas import tpu as pltpu
from jax.experimental.pallas import tpu_sc as plsc

GRID = 20
NC, NS = 2, 16
NW = NC * NS
LANES = 16


def _prep_kernel(haz_ref, surv_ref, cut_ref, tsp_ref,
                 hh_ref, t0_ref, tsm_ref):
    K = cut_ref.shape[1]
    M2 = tsp_ref.shape[1]

    @pl.when(pl.program_id(0) == 0)
    def _tables():
        ts2 = tsp_ref[:, :]
        cnt = jnp.zeros((1, M2), jnp.int32)
        for k in range(K):
            cnt = cnt + (cut_ref[0, k] <= ts2).astype(jnp.int32)
        t0 = cnt - 1
        T0 = jnp.zeros((1, M2), jnp.float32)
        for k in range(K):
            T0 = jnp.where(t0 == k, cut_ref[0, k], T0)
        t0_ref[:, :] = t0
        tsm_ref[:, :] = ts2 - T0

    surv = surv_ref[:, :]
    haz = haz_ref[:, :]
    L = jnp.log(1e-6 + surv)
    d = cut_ref[:, 1:K] - cut_ref[:, 0:K - 1]
    rd = 1.0 / jnp.where(d <= 0.0, 1.0, d)
    hh_ref[:, 0:K - 1] = (L[:, 0:K - 1] - L[:, 1:K]) * rd
    hh_ref[:, K - 1:K] = haz[:, K - 1:K]


def _sc_expand(hh_hbm, surv_hbm, t0_hbm, tsm_hbm,
               hstar_hbm, satt_hbm,
               t0_v, tsm_v, hh_v, surv_v, hst_v, sat_v):
    n = hh_hbm.shape[0]
    M = hstar_hbm.shape[1]
    M2 = t0_hbm.shape[0]
    R = hh_v.shape[0]
    rows_w = n // NW
    wid = lax.axis_index("s") * NC + lax.axis_index("c")
    base = wid * rows_w

    pltpu.sync_copy(t0_hbm, t0_v)
    pltpu.sync_copy(tsm_hbm, tsm_v)

    nj_full = M // LANES
    o_tail = nj_full * LANES

    def chunk(c, carry):
        row0 = base + c * R
        pltpu.sync_copy(hh_hbm.at[pl.ds(row0, R)], hh_v)
        pltpu.sync_copy(surv_hbm.at[pl.ds(row0, R)], surv_v)

        def jstep(jc, carry2):
            o = jc * LANES
            idx = t0_v[pl.ds(o, LANES)]
            nt = 0.0 - tsm_v[pl.ds(o, LANES)]
            for r in range(R):
                rv = jnp.full((LANES,), r, jnp.int32)
                h = plsc.load_gather(hh_v, [rv, idx])
                s0 = plsc.load_gather(surv_v, [rv, idx])
                hst_v[r, pl.ds(o, LANES)] = h
                sat_v[r, pl.ds(o, LANES)] = s0 * jnp.exp(nt * h)
            return carry2

        lax.fori_loop(0, nj_full, jstep, 0)

        colidx = lax.iota(jnp.int32, LANES) + o_tail
        mask = colidx < M
        idx = t0_v[pl.ds(o_tail, LANES)]
        nt = 0.0 - tsm_v[pl.ds(o_tail, LANES)]
        for r in range(R):
            rv = jnp.full((LANES,), r, jnp.int32)
            h = plsc.load_gather(hh_v, [rv, idx])
            s0 = plsc.load_gather(surv_v, [rv, idx])
            plsc.store_scatter(hst_v, [rv, colidx], h, mask=mask)
            plsc.store_scatter(sat_v, [rv, colidx],
                               s0 * jnp.exp(nt * h), mask=mask)

        pltpu.sync_copy(hst_v, hstar_hbm.at[pl.ds(row0, R)])
        pltpu.sync_copy(sat_v, satt_hbm.at[pl.ds(row0, R)])
        return carry

    lax.fori_loop(0, rows_w // R, chunk, 0)


@jax.jit
def kernel(hazard, survival, cut_points):
    n, K = hazard.shape
    M = (K - 1) * GRID
    M2 = ((M + LANES - 1) // LANES) * LANES
    ts = jnp.linspace(cut_points[0], cut_points[-1], M)
    ts_pad = jnp.concatenate(
        [ts, jnp.full((M2 - M,), cut_points[-1], jnp.float32)])

    BN = 2048
    cut2 = cut_points.reshape(1, K)
    tsp2 = ts_pad.reshape(1, M2)

    hh, t0p, tsmp = pl.pallas_call(
        _prep_kernel,
        grid=(n // BN,),
        in_specs=[
            pl.BlockSpec((BN, K), lambda i: (i, 0)),
            pl.BlockSpec((BN, K), lambda i: (i, 0)),
            pl.BlockSpec((1, K), lambda i: (0, 0)),
            pl.BlockSpec((1, M2), lambda i: (0, 0)),
        ],
        out_specs=[
            pl.BlockSpec((BN, K), lambda i: (i, 0)),
            pl.BlockSpec((1, M2), lambda i: (0, 0)),
            pl.BlockSpec((1, M2), lambda i: (0, 0)),
        ],
        out_shape=[
            jax.ShapeDtypeStruct((n, K), jnp.float32),
            jax.ShapeDtypeStruct((1, M2), jnp.int32),
            jax.ShapeDtypeStruct((1, M2), jnp.float32),
        ],
    )(hazard, survival, cut2, tsp2)

    R = 16
    sc_fn = pl.kernel(
        _sc_expand,
        mesh=plsc.VectorSubcoreMesh(core_axis_name="c", subcore_axis_name="s"),
        compiler_params=pltpu.CompilerParams(needs_layout_passes=False),
        out_type=[
            jax.ShapeDtypeStruct((n, M), jnp.float32),
            jax.ShapeDtypeStruct((n, M), jnp.float32),
        ],
        scratch_types=[
            pltpu.VMEM((M2,), jnp.int32),
            pltpu.VMEM((M2,), jnp.float32),
            pltpu.VMEM((R, K), jnp.float32),
            pltpu.VMEM((R, K), jnp.float32),
            pltpu.VMEM((R, M), jnp.float32),
            pltpu.VMEM((R, M), jnp.float32),
        ],
    )
    hstar, satt = sc_fn(hh, survival, t0p.reshape(M2), tsmp.reshape(M2))
    return ts, hstar, satt


# speedup vs baseline: 1.1489x; 1.1489x over previous
import functools

import jax
import jax.numpy as jnp
from jax import lax
from jax.experimental import pallas as pl
from jax.experimental.pallas import tpu as pltpu
from jax.experimental.pallas import tpu_sc as plsc

GRID = 20
NC, NS = 2, 16
NW = NC * NS
LANES = 16


def _prep_kernel(haz_ref, surv_ref, cut_ref, tsp_ref,
                 hh_ref, t0_ref, tsm_ref):
    K = cut_ref.shape[1]
    M2 = tsp_ref.shape[1]

    @pl.when(pl.program_id(0) == 0)
    def _tables():
        ts2 = tsp_ref[:, :]
        cnt = jnp.zeros((1, M2), jnp.int32)
        for k in range(K):
            cnt = cnt + (cut_ref[0, k] <= ts2).astype(jnp.int32)
        t0 = cnt - 1
        T0 = jnp.zeros((1, M2), jnp.float32)
        for k in range(K):
            T0 = jnp.where(t0 == k, cut_ref[0, k], T0)
        t0_ref[:, :] = t0
        tsm_ref[:, :] = ts2 - T0

    surv = surv_ref[:, :]
    haz = haz_ref[:, :]
    L = jnp.log(1e-6 + surv)
    d = cut_ref[:, 1:K] - cut_ref[:, 0:K - 1]
    rd = 1.0 / jnp.where(d <= 0.0, 1.0, d)
    hh_ref[:, 0:K - 1] = (L[:, 0:K - 1] - L[:, 1:K]) * rd
    hh_ref[:, K - 1:K] = haz[:, K - 1:K]


def _sc_expand(hh_hbm, surv_hbm, t0_hbm, tsm_hbm,
               hstar_hbm, satt_hbm,
               t0_v, tsm_v,
               hh_v0, hh_v1, surv_v0, surv_v1,
               hst_v0, hst_v1, sat_v0, sat_v1,
               sem_in0, sem_in1, sem_out0, sem_out1):
    n = hh_hbm.shape[0]
    M = hstar_hbm.shape[1]
    R = hh_v0.shape[0]
    rows_w = n // NW
    nchunks = rows_w // R
    wid = lax.axis_index("s") * NC + lax.axis_index("c")
    base = wid * rows_w

    bufs = ((hh_v0, surv_v0, hst_v0, sat_v0, sem_in0, sem_out0),
            (hh_v1, surv_v1, hst_v1, sat_v1, sem_in1, sem_out1))

    pltpu.sync_copy(t0_hbm, t0_v)
    pltpu.sync_copy(tsm_hbm, tsm_v)

    nj_full = M // LANES
    o_tail = nj_full * LANES
    rvs = [jnp.full((LANES,), r, jnp.int32) for r in range(R)]
    colidx = lax.iota(jnp.int32, LANES) + o_tail
    tailmask = colidx < M

    def in_copies(c, hh_v, surv_v, sem):
        row0 = base + c * R
        return (pltpu.make_async_copy(hh_hbm.at[pl.ds(row0, R)], hh_v, sem),
                pltpu.make_async_copy(surv_hbm.at[pl.ds(row0, R)], surv_v,
                                      sem))

    def out_copies(c, hst_v, sat_v, sem):
        row0 = base + c * R
        return (pltpu.make_async_copy(hst_v, hstar_hbm.at[pl.ds(row0, R)],
                                      sem),
                pltpu.make_async_copy(sat_v, satt_hbm.at[pl.ds(row0, R)],
                                      sem))

    for b in range(2):
        hh_v, surv_v, _, _, sem_in, _ = bufs[b]
        for cp in in_copies(b, hh_v, surv_v, sem_in):
            cp.start()

    def super_chunk(g, carry):
        for b in range(2):
            hh_v, surv_v, hst_v, sat_v, sem_in, sem_out = bufs[b]
            c = g * 2 + b

            for cp in in_copies(c, hh_v, surv_v, sem_in):
                cp.wait()

            @pl.when(c >= 2)
            def _drain():
                for cp in out_copies(c - 2, hst_v, sat_v, sem_out):
                    cp.wait()

            def jstep(jc, carry2):
                o = jc * LANES
                idx = t0_v[pl.ds(o, LANES)]
                nt = 0.0 - tsm_v[pl.ds(o, LANES)]
                for r in range(R):
                    h = plsc.load_gather(hh_v, [rvs[r], idx])
                    s0 = plsc.load_gather(surv_v, [rvs[r], idx])
                    hst_v[r, pl.ds(o, LANES)] = h
                    sat_v[r, pl.ds(o, LANES)] = s0 * jnp.exp(nt * h)
                return carry2

            lax.fori_loop(0, nj_full, jstep, 0)

            idx = t0_v[pl.ds(o_tail, LANES)]
            nt = 0.0 - tsm_v[pl.ds(o_tail, LANES)]
            for r in range(R):
                h = plsc.load_gather(hh_v, [rvs[r], idx])
                s0 = plsc.load_gather(surv_v, [rvs[r], idx])
                plsc.store_scatter(hst_v, [rvs[r], colidx], h, mask=tailmask)
                plsc.store_scatter(sat_v, [rvs[r], colidx],
                                   s0 * jnp.exp(nt * h), mask=tailmask)

            for cp in out_copies(c, hst_v, sat_v, sem_out):
                cp.start()

            @pl.when(c + 2 < nchunks)
            def _prefetch():
                for cp in in_copies(c + 2, hh_v, surv_v, sem_in):
                    cp.start()
        return carry

    lax.fori_loop(0, nchunks // 2, super_chunk, 0)

    for b in range(2):
        _, _, hst_v, sat_v, _, sem_out = bufs[b]
        for cp in out_copies(nchunks - 2 + b, hst_v, sat_v, sem_out):
            cp.wait()


@jax.jit
def kernel(hazard, survival, cut_points):
    n, K = hazard.shape
    M = (K - 1) * GRID
    M2 = ((M + LANES - 1) // LANES) * LANES
    ts = jnp.linspace(cut_points[0], cut_points[-1], M)
    ts_pad = jnp.concatenate(
        [ts, jnp.full((M2 - M,), cut_points[-1], jnp.float32)])

    BN = 2048
    cut2 = cut_points.reshape(1, K)
    tsp2 = ts_pad.reshape(1, M2)

    hh, t0p, tsmp = pl.pallas_call(
        _prep_kernel,
        grid=(n // BN,),
        in_specs=[
            pl.BlockSpec((BN, K), lambda i: (i, 0)),
            pl.BlockSpec((BN, K), lambda i: (i, 0)),
            pl.BlockSpec((1, K), lambda i: (0, 0)),
            pl.BlockSpec((1, M2), lambda i: (0, 0)),
        ],
        out_specs=[
            pl.BlockSpec((BN, K), lambda i: (i, 0)),
            pl.BlockSpec((1, M2), lambda i: (0, 0)),
            pl.BlockSpec((1, M2), lambda i: (0, 0)),
        ],
        out_shape=[
            jax.ShapeDtypeStruct((n, K), jnp.float32),
            jax.ShapeDtypeStruct((1, M2), jnp.int32),
            jax.ShapeDtypeStruct((1, M2), jnp.float32),
        ],
    )(hazard, survival, cut2, tsp2)

    R = 16
    sc_fn = pl.kernel(
        _sc_expand,
        mesh=plsc.VectorSubcoreMesh(core_axis_name="c", subcore_axis_name="s"),
        compiler_params=pltpu.CompilerParams(needs_layout_passes=False),
        out_type=[
            jax.ShapeDtypeStruct((n, M), jnp.float32),
            jax.ShapeDtypeStruct((n, M), jnp.float32),
        ],
        scratch_types=[
            pltpu.VMEM((M2,), jnp.int32),
            pltpu.VMEM((M2,), jnp.float32),
            pltpu.VMEM((R, K), jnp.float32),
            pltpu.VMEM((R, K), jnp.float32),
            pltpu.VMEM((R, K), jnp.float32),
            pltpu.VMEM((R, K), jnp.float32),
            pltpu.VMEM((R, M), jnp.float32),
            pltpu.VMEM((R, M), jnp.float32),
            pltpu.VMEM((R, M), jnp.float32),
            pltpu.VMEM((R, M), jnp.float32),
            pltpu.SemaphoreType.DMA,
            pltpu.SemaphoreType.DMA,
            pltpu.SemaphoreType.DMA,
            pltpu.SemaphoreType.DMA,
        ],
    )
    hstar, satt = sc_fn(hh, survival, t0p.reshape(M2), tsmp.reshape(M2))
    return ts, hstar, satt


# speedup vs baseline: 2.4376x; 2.1216x over previous
import functools

import jax
import jax.numpy as jnp
from jax import lax
from jax.experimental import pallas as pl
from jax.experimental.pallas import tpu as pltpu
from jax.experimental.pallas import tpu_sc as plsc

GRID = 20
NC, NS = 2, 16
NW = NC * NS
LANES = 16


def _prep_kernel(haz_ref, surv_ref, cut_ref, tsp_ref,
                 hh_ref, t0_ref, tsm_ref):
    K = cut_ref.shape[1]
    M2 = tsp_ref.shape[1]

    @pl.when(pl.program_id(0) == 0)
    def _tables():
        ts2 = tsp_ref[:, :]
        cnt = jnp.zeros((1, M2), jnp.int32)
        for k in range(K):
            cnt = cnt + (cut_ref[0, k] <= ts2).astype(jnp.int32)
        t0 = cnt - 1
        T0 = jnp.zeros((1, M2), jnp.float32)
        for k in range(K):
            T0 = jnp.where(t0 == k, cut_ref[0, k], T0)
        t0_ref[:, :] = t0
        tsm_ref[:, :] = ts2 - T0

    surv = surv_ref[:, :]
    haz = haz_ref[:, :]
    L = jnp.log(1e-6 + surv)
    d = cut_ref[:, 1:K] - cut_ref[:, 0:K - 1]
    rd = 1.0 / jnp.where(d <= 0.0, 1.0, d)
    hh_ref[:, 0:K - 1] = (L[:, 0:K - 1] - L[:, 1:K]) * rd
    hh_ref[:, K - 1:K] = haz[:, K - 1:K]


def _sc_expand(hh_hbm, surv_hbm, t0_hbm, tsm_hbm,
               hstar_hbm, satt_hbm,
               t0_v, tsm_v,
               hh_v0, hh_v1, surv_v0, surv_v1,
               hst_v0, hst_v1, sat_v0, sat_v1,
               sem_in0, sem_in1, sem_out0, sem_out1):
    n = hh_hbm.shape[0]
    M = hstar_hbm.shape[1]
    R = hh_v0.shape[0]
    rows_w = n // NW
    nchunks = rows_w // R
    wid = lax.axis_index("s") * NC + lax.axis_index("c")
    base = wid * rows_w

    bufs = ((hh_v0, surv_v0, hst_v0, sat_v0, sem_in0, sem_out0),
            (hh_v1, surv_v1, hst_v1, sat_v1, sem_in1, sem_out1))

    pltpu.sync_copy(t0_hbm, t0_v)
    pltpu.sync_copy(tsm_hbm, tsm_v)

    nj_full = M // LANES
    o_tail = nj_full * LANES
    rvs = [jnp.full((LANES,), r, jnp.int32) for r in range(R)]
    colidx = lax.iota(jnp.int32, LANES) + o_tail
    tailmask = colidx < M

    def in_copies(c, hh_v, surv_v, sem):
        row0 = base + c * R
        return (pltpu.make_async_copy(hh_hbm.at[pl.ds(row0, R)], hh_v, sem),
                pltpu.make_async_copy(surv_hbm.at[pl.ds(row0, R)], surv_v,
                                      sem))

    def out_copies(c, hst_v, sat_v, sem):
        row0 = base + c * R
        return (pltpu.make_async_copy(hst_v, hstar_hbm.at[pl.ds(row0, R)],
                                      sem),
                pltpu.make_async_copy(sat_v, satt_hbm.at[pl.ds(row0, R)],
                                      sem))

    for b in range(2):
        hh_v, surv_v, _, _, sem_in, _ = bufs[b]
        for cp in in_copies(b, hh_v, surv_v, sem_in):
            cp.start()

    def super_chunk(g, carry):
        for b in range(2):
            hh_v, surv_v, hst_v, sat_v, sem_in, sem_out = bufs[b]
            c = g * 2 + b

            for cp in in_copies(c, hh_v, surv_v, sem_in):
                cp.wait()

            @pl.when(c >= 2)
            def _drain():
                for cp in out_copies(c - 2, hst_v, sat_v, sem_out):
                    cp.wait()

            @plsc.parallel_loop(0, nj_full, unroll=2)
            def jstep(jc):
                o = jc * LANES
                idx = t0_v[pl.ds(o, LANES)]
                nt = 0.0 - tsm_v[pl.ds(o, LANES)]
                for r in range(R):
                    h = plsc.load_gather(hh_v, [rvs[r], idx])
                    s0 = plsc.load_gather(surv_v, [rvs[r], idx])
                    hst_v[r, pl.ds(o, LANES)] = h
                    sat_v[r, pl.ds(o, LANES)] = s0 * jnp.exp(nt * h)

            idx = t0_v[pl.ds(o_tail, LANES)]
            nt = 0.0 - tsm_v[pl.ds(o_tail, LANES)]
            for r in range(R):
                h = plsc.load_gather(hh_v, [rvs[r], idx])
                s0 = plsc.load_gather(surv_v, [rvs[r], idx])
                plsc.store_scatter(hst_v, [rvs[r], colidx], h, mask=tailmask)
                plsc.store_scatter(sat_v, [rvs[r], colidx],
                                   s0 * jnp.exp(nt * h), mask=tailmask)

            for cp in out_copies(c, hst_v, sat_v, sem_out):
                cp.start()

            @pl.when(c + 2 < nchunks)
            def _prefetch():
                for cp in in_copies(c + 2, hh_v, surv_v, sem_in):
                    cp.start()
        return carry

    lax.fori_loop(0, nchunks // 2, super_chunk, 0)

    for b in range(2):
        _, _, hst_v, sat_v, _, sem_out = bufs[b]
        for cp in out_copies(nchunks - 2 + b, hst_v, sat_v, sem_out):
            cp.wait()


@jax.jit
def kernel(hazard, survival, cut_points):
    n, K = hazard.shape
    M = (K - 1) * GRID
    M2 = ((M + LANES - 1) // LANES) * LANES
    ts = jnp.linspace(cut_points[0], cut_points[-1], M)
    ts_pad = jnp.concatenate(
        [ts, jnp.full((M2 - M,), cut_points[-1], jnp.float32)])

    BN = 2048
    cut2 = cut_points.reshape(1, K)
    tsp2 = ts_pad.reshape(1, M2)

    hh, t0p, tsmp = pl.pallas_call(
        _prep_kernel,
        grid=(n // BN,),
        in_specs=[
            pl.BlockSpec((BN, K), lambda i: (i, 0)),
            pl.BlockSpec((BN, K), lambda i: (i, 0)),
            pl.BlockSpec((1, K), lambda i: (0, 0)),
            pl.BlockSpec((1, M2), lambda i: (0, 0)),
        ],
        out_specs=[
            pl.BlockSpec((BN, K), lambda i: (i, 0)),
            pl.BlockSpec((1, M2), lambda i: (0, 0)),
            pl.BlockSpec((1, M2), lambda i: (0, 0)),
        ],
        out_shape=[
            jax.ShapeDtypeStruct((n, K), jnp.float32),
            jax.ShapeDtypeStruct((1, M2), jnp.int32),
            jax.ShapeDtypeStruct((1, M2), jnp.float32),
        ],
    )(hazard, survival, cut2, tsp2)

    R = 16
    sc_fn = pl.kernel(
        _sc_expand,
        mesh=plsc.VectorSubcoreMesh(core_axis_name="c", subcore_axis_name="s"),
        compiler_params=pltpu.CompilerParams(needs_layout_passes=False),
        out_type=[
            jax.ShapeDtypeStruct((n, M), jnp.float32),
            jax.ShapeDtypeStruct((n, M), jnp.float32),
        ],
        scratch_types=[
            pltpu.VMEM((M2,), jnp.int32),
            pltpu.VMEM((M2,), jnp.float32),
            pltpu.VMEM((R, K), jnp.float32),
            pltpu.VMEM((R, K), jnp.float32),
            pltpu.VMEM((R, K), jnp.float32),
            pltpu.VMEM((R, K), jnp.float32),
            pltpu.VMEM((R, M), jnp.float32),
            pltpu.VMEM((R, M), jnp.float32),
            pltpu.VMEM((R, M), jnp.float32),
            pltpu.VMEM((R, M), jnp.float32),
            pltpu.SemaphoreType.DMA,
            pltpu.SemaphoreType.DMA,
            pltpu.SemaphoreType.DMA,
            pltpu.SemaphoreType.DMA,
        ],
    )
    hstar, satt = sc_fn(hh, survival, t0p.reshape(M2), tsmp.reshape(M2))
    return ts, hstar, satt


# speedup vs baseline: 3.5057x; 1.4382x over previous
import jax
import jax.numpy as jnp
from jax.experimental import pallas as pl
from jax.experimental.pallas import tpu as pltpu

GRID = 20


def _interp_kernel(haz_ref, surv_ref, cut_ref, ts_ref,
                   hstar_hbm, satt_hbm,
                   w1_ref, w2_ref, tsmT0_ref,
                   hbuf0, hbuf1, sbuf0, sbuf1,
                   semh0, semh1, sems0, sems1):
    K = cut_ref.shape[1]
    M = ts_ref.shape[1]
    BN = haz_ref.shape[0]
    i = pl.program_id(0)
    nsteps = pl.num_programs(0)

    @pl.when(i == 0)
    def _build_tables():
        ts2 = ts_ref[:, :]
        cnt = jnp.zeros((1, M), jnp.int32)
        for k in range(K):
            cnt = cnt + (cut_ref[0, k] <= ts2).astype(jnp.int32)
        t0 = cnt - 1
        t1 = jnp.where(cnt == K, K - 1, cnt)
        T0 = jnp.zeros((1, M), jnp.float32)
        T1 = jnp.zeros((1, M), jnp.float32)
        for k in range(K):
            ck = cut_ref[0, k]
            T0 = jnp.where(t0 == k, ck, T0)
            T1 = jnp.where(t1 == k, ck, T1)
        dT = T1 - T0
        neg = dT <= 0.0
        rdT = 1.0 / jnp.where(neg, 1.0, dT)
        tsmT0_ref[:, :] = ts2 - T0
        ki = jax.lax.broadcasted_iota(jnp.int32, (K, M), 0)
        p0 = (ki == t0).astype(jnp.float32)
        p1 = (ki == t1).astype(jnp.float32)
        pdr = (p0 - p1) * rdT
        p0n = p0 * jnp.where(neg, 1.0, 0.0)
        pdr_hi = pdr.astype(jnp.bfloat16)
        pdr_lo = (pdr - pdr_hi.astype(jnp.float32)).astype(jnp.bfloat16)
        p0n_bf = p0n.astype(jnp.bfloat16)
        p0_bf = p0.astype(jnp.bfloat16)
        w1_ref[0 * K:1 * K, :] = pdr_hi
        w1_ref[1 * K:2 * K, :] = pdr_hi
        w1_ref[2 * K:3 * K, :] = pdr_lo
        w1_ref[3 * K:4 * K, :] = p0n_bf
        w1_ref[4 * K:5 * K, :] = p0n_bf
        w2_ref[0 * K:1 * K, :] = p0_bf
        w2_ref[1 * K:2 * K, :] = p0_bf

    surv = surv_ref[:, :]
    haz = haz_ref[:, :]
    logs = jnp.log(1e-6 + surv)

    def split(x):
        hi = x.astype(jnp.bfloat16)
        lo = (x - hi.astype(jnp.float32)).astype(jnp.bfloat16)
        return hi, lo

    s_hi, s_lo = split(surv)
    h_hi, h_lo = split(haz)
    l_hi, l_lo = split(logs)

    lhs1 = jnp.concatenate([l_hi, l_lo, l_hi, h_hi, h_lo], axis=1)
    lhs2 = jnp.concatenate([s_hi, s_lo], axis=1)

    hstar = jnp.dot(lhs1, w1_ref[:, :], preferred_element_type=jnp.float32)
    S0 = jnp.dot(lhs2, w2_ref[:, :], preferred_element_type=jnp.float32)
    satt = S0 * jnp.exp(-tsmT0_ref[:, :] * hstar)

    def copies(hb, sb, row_start, sh, ss):
        return (pltpu.make_async_copy(hb, hstar_hbm.at[pl.ds(row_start, BN)],
                                      sh),
                pltpu.make_async_copy(sb, satt_hbm.at[pl.ds(row_start, BN)],
                                      ss))

    def ship(hb, sb, sh, ss):
        @pl.when(i >= 2)
        def _drain():
            for cp in copies(hb, sb, (i - 2) * BN, sh, ss):
                cp.wait()
        hb[:, :] = hstar
        sb[:, :] = satt
        for cp in copies(hb, sb, i * BN, sh, ss):
            cp.start()

    @pl.when(i % 2 == 0)
    def _even():
        ship(hbuf0, sbuf0, semh0, sems0)

    @pl.when(i % 2 == 1)
    def _odd():
        ship(hbuf1, sbuf1, semh1, sems1)

    @pl.when(i == nsteps - 1)
    def _epilogue():
        for cp in copies(hbuf0, sbuf0, (nsteps - 2) * BN, semh0, sems0):
            cp.wait()
        for cp in copies(hbuf1, sbuf1, (nsteps - 1) * BN, semh1, sems1):
            cp.wait()


@jax.jit
def kernel(hazard, survival, cut_points):
    n, K = hazard.shape
    M = (K - 1) * GRID
    ts = jnp.linspace(cut_points[0], cut_points[-1], M)

    BN = 512
    grid = (n // BN,)
    cut2 = cut_points.reshape(1, K)
    ts2 = ts.reshape(1, M)

    hstar, satt = pl.pallas_call(
        _interp_kernel,
        grid=grid,
        in_specs=[
            pl.BlockSpec((BN, K), lambda i: (i, 0)),
            pl.BlockSpec((BN, K), lambda i: (i, 0)),
            pl.BlockSpec((1, K), lambda i: (0, 0)),
            pl.BlockSpec((1, M), lambda i: (0, 0)),
        ],
        out_specs=[
            pl.BlockSpec(memory_space=pltpu.HBM),
            pl.BlockSpec(memory_space=pltpu.HBM),
        ],
        out_shape=[
            jax.ShapeDtypeStruct((n, M), jnp.float32),
            jax.ShapeDtypeStruct((n, M), jnp.float32),
        ],
        scratch_shapes=[
            pltpu.VMEM((5 * K, M), jnp.bfloat16),
            pltpu.VMEM((2 * K, M), jnp.bfloat16),
            pltpu.VMEM((1, M), jnp.float32),
            pltpu.VMEM((BN, M), jnp.float32),
            pltpu.VMEM((BN, M), jnp.float32),
            pltpu.VMEM((BN, M), jnp.float32),
            pltpu.VMEM((BN, M), jnp.float32),
            pltpu.SemaphoreType.DMA,
            pltpu.SemaphoreType.DMA,
            pltpu.SemaphoreType.DMA,
            pltpu.SemaphoreType.DMA,
        ],
    )(hazard, survival, cut2, ts2)
    return ts, hstar, satt
